# baseline (speedup 1.0000x reference)
import jax
import jax.numpy as jnp
from jax import lax
from jax.experimental import pallas as pl
from jax.experimental.pallas import tpu as pltpu
from jax.experimental.pallas import tpu_sc as plsc

B = 1024
S = 200
D = 64
NC = 2
NS = 16
NW = NC * NS
N = B * S
PER_W = N // NW
CHUNK = 320
NCHUNK = PER_W // CHUNK


def _body(tok_hbm, pos_hbm, e_hbm, p_hbm, out_hbm,
          idx_v, pos_v, rows0, rows1, p_sh,
          semG0, semG1, semP, semS0, semS1):
    cid = lax.axis_index("c")
    sid = lax.axis_index("s")
    wid = sid * NC + cid
    base_w = wid * PER_W

    @pl.when(sid == 0)
    def _():
        pltpu.sync_copy(p_hbm, p_sh)

    plsc.subcore_barrier()

    pltpu.sync_copy(tok_hbm.at[pl.ds(base_w, PER_W)], idx_v)
    pltpu.sync_copy(pos_hbm.at[pl.ds(base_w, PER_W)], pos_v)

    rows = [rows0, rows1]
    semG = [semG0, semG1]
    semS = [semS0, semS1]

    def gather(c, b):
        def q_body(q, _):
            toks = idx_v[pl.ds(c * CHUNK + q * 16, 16)]
            for l in range(16):
                pltpu.async_copy(
                    e_hbm.at[toks[l]],
                    rows[b].at[q * 16 + l, pl.ds(0, D)], semG[b])
            return 0
        lax.fori_loop(0, CHUNK // 16, q_body, 0, unroll=False)

    def gather_wait(b):
        pltpu.make_async_copy(
            out_hbm.at[pl.ds(0, CHUNK // 2)],
            rows[b].at[pl.ds(0, CHUNK // 2)], semG[b]).wait()

    def padd(c, b):
        return pltpu.async_copy(
            p_sh.at[pos_v.at[pl.ds(c * CHUNK, CHUNK)]], rows[b],
            semP, add=True)

    def store(c, b):
        return pltpu.async_copy(
            rows[b], out_hbm.at[pl.ds(base_w + c * CHUNK, CHUNK)], semS[b])

    def store_wait(b):
        pltpu.make_async_copy(
            rows[b], out_hbm.at[pl.ds(0, CHUNK)], semS[b]).wait()

    gather(0, 0)

    def c_body(g, _):
        for b in range(2):
            c = g * 2 + b
            gather_wait(b)
            pa = padd(c, b)

            @pl.when(c + 1 < NCHUNK)
            def _():
                @pl.when(c >= 1)
                def _():
                    store_wait((b + 1) % 2)
                gather(c + 1, (b + 1) % 2)

            pa.wait()
            store(c, b)
        return 0

    lax.fori_loop(0, NCHUNK // 2, c_body, 0, unroll=False)
    store_wait(0)
    store_wait(1)


def kernel(tokens, E, P):
    p128 = jnp.pad(P, ((0, 0), (0, 64)))
    mesh = plsc.VectorSubcoreMesh(
        core_axis_name="c", subcore_axis_name="s", num_cores=NC, num_subcores=NS
    )
    run = pl.kernel(
        _body,
        out_type=jax.ShapeDtypeStruct((N, 128), jnp.float32),
        mesh=mesh,
        compiler_params=pltpu.CompilerParams(
            use_tc_tiling_on_sc=True, needs_layout_passes=False),
        scratch_types=[
            pltpu.VMEM((PER_W,), jnp.int32),
            pltpu.VMEM((PER_W,), jnp.int32),
            pltpu.VMEM((CHUNK, 128), jnp.float32),
            pltpu.VMEM((CHUNK, 128), jnp.float32),
            pltpu.VMEM_SHARED((S, 128), jnp.float32),
            pltpu.SemaphoreType.DMA,
            pltpu.SemaphoreType.DMA,
            pltpu.SemaphoreType.DMA,
            pltpu.SemaphoreType.DMA,
            pltpu.SemaphoreType.DMA,
        ],
    )
    pos = jnp.broadcast_to(jnp.arange(S, dtype=jnp.int32)[None, :], (B, S))
    out = run(tokens.reshape(N), pos.reshape(N), E, p128)
    return out[:, :D].reshape(B, S, D)
